# baseline (speedup 1.0000x reference)
import functools

import jax
import jax.numpy as jnp
from jax import lax
from jax.experimental import pallas as pl
from jax.experimental.pallas import tpu as pltpu
from jax.experimental.pallas import tpu_sc as plsc


def _lane_permute(x, perm):
    dnums = lax.GatherDimensionNumbers(
        offset_dims=(), collapsed_slice_dims=(0,), start_index_map=(0,))
    return lax.gather(x, perm[:, None], dnums, (1,),
                      mode=lax.GatherScatterMode.PROMISE_IN_BOUNDS)


def _make_sc_kernel(N, D, CH):
    NW = 32
    TPW = N // NW
    NCH = TPW // CH
    NV = D // 16
    CH_A = 104
    CH_B = CH - CH_A

    mesh = plsc.VectorSubcoreMesh(core_axis_name="c", subcore_axis_name="s")

    @functools.partial(
        pl.kernel,
        mesh=mesh,
        out_type=jax.ShapeDtypeStruct((N, D), jnp.float32),
        scratch_types=[
            pltpu.VMEM((TPW,), jnp.int32),
            pltpu.VMEM((TPW + 16,), jnp.int32),
            pltpu.VMEM((2 * CH, D), jnp.float32),
            pltpu.VMEM((2, D), jnp.float32),
            pltpu.VMEM((CH, D), jnp.float32),
            pltpu.VMEM((CH, D), jnp.float32),
            pltpu.VMEM((D,), jnp.float32),
            pltpu.VMEM((D,), jnp.float32),
            pltpu.SemaphoreType.DMA,
            pltpu.SemaphoreType.DMA,
            pltpu.SemaphoreType.DMA,
            pltpu.SemaphoreType.DMA,
        ],
    )
    def emb_kernel(idx_hbm, seg_hbm, tok_hbm, segtab_hbm, pos_hbm,
                   g_hbm, b_hbm, out_hbm,
                   idx_all, seg_all, posbuf, segrows, rows0, rows1,
                   gvec, bvec, gsem0, gsem1, wsem0, wsem1):
        wid = lax.axis_index("s") * 2 + lax.axis_index("c")
        base = wid * TPW

        pltpu.sync_copy(idx_hbm.at[pl.ds(base, TPW)], idx_all)
        pltpu.sync_copy(seg_hbm.at[pl.ds(base, TPW)], seg_all.at[pl.ds(0, TPW)])
        pltpu.sync_copy(pos_hbm.at[pl.ds(0, CH)], posbuf.at[pl.ds(0, CH)])
        pltpu.sync_copy(pos_hbm.at[pl.ds(0, CH)], posbuf.at[pl.ds(CH, CH)])
        pltpu.sync_copy(segtab_hbm.at[pl.ds(0, 2)], segrows)
        pltpu.sync_copy(g_hbm, gvec)
        pltpu.sync_copy(b_hbm, bvec)

        @plsc.parallel_loop(0, CH, 1, unroll=2)
        def add_seg(r):
            for j in range(NV):
                sl = pl.ds(16 * j, 16)
                posbuf[r, sl] = posbuf[r, sl] + segrows[0, sl]
                posbuf[CH + r, sl] = posbuf[CH + r, sl] + segrows[1, sl]

        def issue_gather(c, buf, sem):
            pltpu.async_copy(tok_hbm.at[idx_all.at[pl.ds(c * CH, CH_A)]],
                             buf.at[pl.ds(0, CH_A)], sem)
            pltpu.async_copy(tok_hbm.at[idx_all.at[pl.ds(c * CH + CH_A, CH_B)]],
                             buf.at[pl.ds(CH_A, CH_B)], sem)

        def wait_gather(buf, sem):
            pltpu.make_async_copy(out_hbm.at[pl.ds(0, CH)], buf, sem).wait()

        def issue_write(c, buf, sem):
            pltpu.async_copy(buf, out_hbm.at[pl.ds(base + c * CH, CH)], sem)

        def wait_write(buf, sem):
            pltpu.make_async_copy(buf, out_hbm.at[pl.ds(0, CH)], sem).wait()

        def _ln_stats(xs):
            acc = xs[0]
            sq = xs[0] * xs[0]
            for j in range(1, NV):
                acc = acc + xs[j]
                sq = sq + xs[j] * xs[j]
            for s in (8, 4, 2, 1):
                perm = jnp.arange(16, dtype=jnp.int32) ^ s
                acc = acc + _lane_permute(acc, perm)
                sq = sq + _lane_permute(sq, perm)
            mean = acc * (1.0 / D)
            v = sq * (1.0 / D) - mean * mean + 1e-5
            iv = lax.bitcast_convert_type(v, jnp.int32)
            iv = jnp.int32(0x5F3759DF) - (iv >> 1)
            y = lax.bitcast_convert_type(iv, jnp.float32)
            vh = -0.5 * v
            for _ in range(2):
                y = y * (1.5 + vh * (y * y))
            return y, mean * y

        def ln_rows(buf, c):
            cbase = c * CH

            @plsc.parallel_loop(0, CH, 2, unroll=2)
            def ln_row_pair(k):
                k1 = k + 1
                sv0 = seg_all[pl.ds(cbase + k, 16)][0]
                sv1 = seg_all[pl.ds(cbase + k1, 16)][0]
                pr0 = sv0 * CH + k
                pr1 = sv1 * CH + k1
                xs0, xs1 = [], []
                for j in range(NV):
                    sl = pl.ds(16 * j, 16)
                    xs0.append(buf[k, sl] + posbuf[pr0, sl])
                    xs1.append(buf[k1, sl] + posbuf[pr1, sl])
                y0, minv0 = _ln_stats(xs0)
                y1, minv1 = _ln_stats(xs1)
                for j in range(NV):
                    sl = pl.ds(16 * j, 16)
                    g = gvec[sl]
                    b = bvec[sl]
                    buf[k, sl] = (xs0[j] * y0 - minv0) * g + b
                    buf[k1, sl] = (xs1[j] * y1 - minv1) * g + b

        issue_gather(0, rows0, gsem0)
        issue_gather(1, rows1, gsem1)

        def stage(c, buf, gsem, wsem, prefetch):
            wait_gather(buf, gsem)
            ln_rows(buf, c)
            issue_write(c, buf, wsem)
            if prefetch:
                wait_write(buf, wsem)
                issue_gather(c + 2, buf, gsem)

        def pair(i, carry):
            stage(2 * i, rows0, gsem0, wsem0, True)
            stage(2 * i + 1, rows1, gsem1, wsem1, True)
            return carry

        lax.fori_loop(0, NCH // 2 - 1, pair, 0)
        stage(NCH - 2, rows0, gsem0, wsem0, False)
        stage(NCH - 1, rows1, gsem1, wsem1, False)
        wait_write(rows0, wsem0)
        wait_write(rows1, wsem1)

    return emb_kernel


def kernel(input_tensor, segment_tensor, token_table, segment_table,
           position_table, gamma, beta):
    B, L = input_tensor.shape
    V, D = token_table.shape
    N = B * L
    idx = input_tensor.reshape(N).astype(jnp.int32)
    sidx = segment_tensor.reshape(N).astype(jnp.int32)
    emb = _make_sc_kernel(N, D, L)
    out = emb(idx, sidx, token_table, segment_table, position_table,
              gamma, beta)
    return out.reshape(B, L, D)


# speedup vs baseline: 1.0389x; 1.0389x over previous
import functools

import jax
import jax.numpy as jnp
from jax import lax
from jax.experimental import pallas as pl
from jax.experimental.pallas import tpu as pltpu
from jax.experimental.pallas import tpu_sc as plsc


def _lane_permute(x, perm):
    dnums = lax.GatherDimensionNumbers(
        offset_dims=(), collapsed_slice_dims=(0,), start_index_map=(0,))
    return lax.gather(x, perm[:, None], dnums, (1,),
                      mode=lax.GatherScatterMode.PROMISE_IN_BOUNDS)


def _make_sc_kernel(N, D, CH):
    NW = 32
    TPW = N // NW
    NCH = TPW // CH
    NV = D // 16
    NI = (CH + 15) // 16
    CH_A = 104
    CH_B = CH - CH_A

    mesh = plsc.VectorSubcoreMesh(core_axis_name="c", subcore_axis_name="s")

    @functools.partial(
        pl.kernel,
        mesh=mesh,
        out_type=jax.ShapeDtypeStruct((N, D), jnp.float32),
        scratch_types=[
            pltpu.VMEM((TPW,), jnp.int32),
            pltpu.VMEM((TPW + 16,), jnp.int32),
            pltpu.VMEM((16 * NI,), jnp.int32),
            pltpu.VMEM((16 * NI,), jnp.int32),
            pltpu.VMEM((CH, D), jnp.float32),
            pltpu.VMEM((CH, D), jnp.float32),
            pltpu.VMEM((D,), jnp.float32),
            pltpu.VMEM((D,), jnp.float32),
            pltpu.SemaphoreType.DMA,
            pltpu.SemaphoreType.DMA,
            pltpu.SemaphoreType.DMA,
            pltpu.SemaphoreType.DMA,
            pltpu.SemaphoreType.DMA,
            pltpu.SemaphoreType.DMA,
        ],
    )
    def emb_kernel(idx_hbm, seg_hbm, tok_hbm, ps_hbm, g_hbm, b_hbm, out_hbm,
                   idx_all, seg_all, pridx0, pridx1, rows0, rows1,
                   gvec, bvec, gsem0, gsem1, psem0, psem1, wsem0, wsem1):
        wid = lax.axis_index("s") * 2 + lax.axis_index("c")
        base = wid * TPW

        pltpu.sync_copy(idx_hbm.at[pl.ds(base, TPW)], idx_all)
        pltpu.sync_copy(seg_hbm.at[pl.ds(base, TPW)], seg_all.at[pl.ds(0, TPW)])
        pltpu.sync_copy(g_hbm, gvec)
        pltpu.sync_copy(b_hbm, bvec)

        def compute_pridx(c, pbuf):
            cbase = c * CH

            @plsc.parallel_loop(0, NI, 1, unroll=2)
            def _(m):
                sl = pl.ds(16 * m, 16)
                sv = seg_all[pl.ds(cbase + 16 * m, 16)]
                pbuf[sl] = sv * CH + (lax.iota(jnp.int32, 16) + 16 * m)

        def issue_posseg(pbuf, buf, sem):
            pltpu.async_copy(ps_hbm.at[pbuf.at[pl.ds(0, CH_A)]],
                             buf.at[pl.ds(0, CH_A)], sem)
            pltpu.async_copy(ps_hbm.at[pbuf.at[pl.ds(CH_A, CH_B)]],
                             buf.at[pl.ds(CH_A, CH_B)], sem)

        def issue_tokadd(c, buf, sem):
            pltpu.async_copy(tok_hbm.at[idx_all.at[pl.ds(c * CH, CH_A)]],
                             buf.at[pl.ds(0, CH_A)], sem, add=True)
            pltpu.async_copy(tok_hbm.at[idx_all.at[pl.ds(c * CH + CH_A, CH_B)]],
                             buf.at[pl.ds(CH_A, CH_B)], sem, add=True)

        def wait_full(buf, sem):
            pltpu.make_async_copy(out_hbm.at[pl.ds(0, CH)], buf, sem).wait()

        def issue_write(c, buf, sem):
            pltpu.async_copy(buf, out_hbm.at[pl.ds(base + c * CH, CH)], sem)

        def wait_write(buf, sem):
            pltpu.make_async_copy(buf, out_hbm.at[pl.ds(0, CH)], sem).wait()

        def _ln_stats(xs):
            acc = xs[0]
            sq = xs[0] * xs[0]
            for j in range(1, NV):
                acc = acc + xs[j]
                sq = sq + xs[j] * xs[j]
            for s in (8, 4, 2, 1):
                perm = jnp.arange(16, dtype=jnp.int32) ^ s
                acc = acc + _lane_permute(acc, perm)
                sq = sq + _lane_permute(sq, perm)
            mean = acc * (1.0 / D)
            v = sq * (1.0 / D) - mean * mean + 1e-5
            iv = lax.bitcast_convert_type(v, jnp.int32)
            iv = jnp.int32(0x5F3759DF) - (iv >> 1)
            y = lax.bitcast_convert_type(iv, jnp.float32)
            vh = -0.5 * v
            for _ in range(2):
                y = y * (1.5 + vh * (y * y))
            return y, mean * y

        def ln_rows(buf):
            @plsc.parallel_loop(0, CH, 2, unroll=4)
            def ln_row_pair(k):
                k1 = k + 1
                xs0 = [buf[k, pl.ds(16 * j, 16)] for j in range(NV)]
                xs1 = [buf[k1, pl.ds(16 * j, 16)] for j in range(NV)]
                y0, minv0 = _ln_stats(xs0)
                y1, minv1 = _ln_stats(xs1)
                for j in range(NV):
                    sl = pl.ds(16 * j, 16)
                    g = gvec[sl]
                    b = bvec[sl]
                    buf[k, sl] = (xs0[j] * y0 - minv0) * g + b
                    buf[k1, sl] = (xs1[j] * y1 - minv1) * g + b

        def prep(c, pbuf, buf, gsem, psem):
            compute_pridx(c, pbuf)
            issue_posseg(pbuf, buf, psem)
            wait_full(buf, psem)
            issue_tokadd(c, buf, gsem)

        def stage(c, pbuf, buf, gsem, psem, wsem, prefetch):
            wait_full(buf, gsem)
            ln_rows(buf)
            issue_write(c, buf, wsem)
            if prefetch:
                wait_write(buf, wsem)
                prep(c + 2, pbuf, buf, gsem, psem)

        prep(0, pridx0, rows0, gsem0, psem0)
        prep(1, pridx1, rows1, gsem1, psem1)

        def pair(i, carry):
            stage(2 * i, pridx0, rows0, gsem0, psem0, wsem0, True)
            stage(2 * i + 1, pridx1, rows1, gsem1, psem1, wsem1, True)
            return carry

        lax.fori_loop(0, NCH // 2 - 1, pair, 0)
        stage(NCH - 2, pridx0, rows0, gsem0, psem0, wsem0, False)
        stage(NCH - 1, pridx1, rows1, gsem1, psem1, wsem1, False)
        wait_write(rows0, wsem0)
        wait_write(rows1, wsem1)

    return emb_kernel


def kernel(input_tensor, segment_tensor, token_table, segment_table,
           position_table, gamma, beta):
    B, L = input_tensor.shape
    V, D = token_table.shape
    N = B * L
    idx = input_tensor.reshape(N).astype(jnp.int32)
    sidx = segment_tensor.reshape(N).astype(jnp.int32)
    posseg = (segment_table[:2][:, None, :]
              + position_table[:L][None, :, :]).reshape(2 * L, D)
    emb = _make_sc_kernel(N, D, L)
    out = emb(idx, sidx, token_table, posseg, gamma, beta)
    return out.reshape(B, L, D)


# speedup vs baseline: 1.5264x; 1.4692x over previous
import functools

import jax
import jax.numpy as jnp
from jax import lax
from jax.experimental import pallas as pl
from jax.experimental.pallas import tpu as pltpu
from jax.experimental.pallas import tpu_sc as plsc


def _lane_permute(x, perm):
    dnums = lax.GatherDimensionNumbers(
        offset_dims=(), collapsed_slice_dims=(0,), start_index_map=(0,))
    return lax.gather(x, perm[:, None], dnums, (1,),
                      mode=lax.GatherScatterMode.PROMISE_IN_BOUNDS)


def _make_sc_kernel(N, D, CH):
    NW = 32
    TPW = N // NW
    NCH = TPW // CH
    NV = D // 16
    CH_A = 104
    CH_B = CH - CH_A

    mesh = plsc.VectorSubcoreMesh(core_axis_name="c", subcore_axis_name="s")

    @functools.partial(
        pl.kernel,
        mesh=mesh,
        out_type=jax.ShapeDtypeStruct((N, D), jnp.float32),
        scratch_types=[
            pltpu.VMEM((TPW,), jnp.int32),
            pltpu.VMEM((TPW + 16,), jnp.int32),
            pltpu.VMEM((2 * CH, D), jnp.float32),
            pltpu.VMEM((2, D), jnp.float32),
            pltpu.VMEM((CH, D), jnp.float32),
            pltpu.VMEM((CH, D), jnp.float32),
            pltpu.VMEM((D,), jnp.float32),
            pltpu.VMEM((D,), jnp.float32),
            pltpu.SemaphoreType.DMA,
            pltpu.SemaphoreType.DMA,
            pltpu.SemaphoreType.DMA,
            pltpu.SemaphoreType.DMA,
        ],
    )
    def emb_kernel(idx_hbm, seg_hbm, tok_hbm, segtab_hbm, pos_hbm,
                   g_hbm, b_hbm, out_hbm,
                   idx_all, seg_all, posbuf, segrows, rows0, rows1,
                   gvec, bvec, gsem0, gsem1, wsem0, wsem1):
        wid = lax.axis_index("s") * 2 + lax.axis_index("c")
        base = wid * TPW

        pltpu.sync_copy(idx_hbm.at[pl.ds(base, TPW)], idx_all)
        pltpu.sync_copy(seg_hbm.at[pl.ds(base, TPW)], seg_all.at[pl.ds(0, TPW)])
        pltpu.sync_copy(pos_hbm.at[pl.ds(0, CH)], posbuf.at[pl.ds(0, CH)])
        pltpu.sync_copy(pos_hbm.at[pl.ds(0, CH)], posbuf.at[pl.ds(CH, CH)])
        pltpu.sync_copy(segtab_hbm.at[pl.ds(0, 2)], segrows)
        pltpu.sync_copy(g_hbm, gvec)
        pltpu.sync_copy(b_hbm, bvec)

        @plsc.parallel_loop(0, CH, 1, unroll=2)
        def add_seg(r):
            for j in range(NV):
                sl = pl.ds(16 * j, 16)
                posbuf[r, sl] = posbuf[r, sl] + segrows[0, sl]
                posbuf[CH + r, sl] = posbuf[CH + r, sl] + segrows[1, sl]

        def issue_gather(c, buf, sem):
            pltpu.async_copy(tok_hbm.at[idx_all.at[pl.ds(c * CH, CH_A)]],
                             buf.at[pl.ds(0, CH_A)], sem)
            pltpu.async_copy(tok_hbm.at[idx_all.at[pl.ds(c * CH + CH_A, CH_B)]],
                             buf.at[pl.ds(CH_A, CH_B)], sem)

        def wait_gather(buf, sem):
            pltpu.make_async_copy(out_hbm.at[pl.ds(0, CH)], buf, sem).wait()

        def issue_write(c, buf, sem):
            pltpu.async_copy(buf, out_hbm.at[pl.ds(base + c * CH, CH)], sem)

        def wait_write(buf, sem):
            pltpu.make_async_copy(buf, out_hbm.at[pl.ds(0, CH)], sem).wait()

        def _ln_stats(xs):
            acc = xs[0]
            sq = xs[0] * xs[0]
            for j in range(1, NV):
                acc = acc + xs[j]
                sq = sq + xs[j] * xs[j]
            for s in (8, 4, 2, 1):
                perm = jnp.arange(16, dtype=jnp.int32) ^ s
                acc = acc + _lane_permute(acc, perm)
                sq = sq + _lane_permute(sq, perm)
            mean = acc * (1.0 / D)
            v = sq * (1.0 / D) - mean * mean + 1e-5
            iv = lax.bitcast_convert_type(v, jnp.int32)
            iv = jnp.int32(0x5F3759DF) - (iv >> 1)
            y = lax.bitcast_convert_type(iv, jnp.float32)
            vh = -0.5 * v
            for _ in range(2):
                y = y * (1.5 + vh * (y * y))
            return y, mean * y

        RG = 4

        def ln_rows(buf, c):
            cbase = c * CH

            @plsc.parallel_loop(0, CH, RG, unroll=2)
            def ln_row_group(k):
                ks, xs, ys, minvs = [], [], [], []
                for r in range(RG):
                    kr = k + r
                    sv = seg_all[pl.ds(cbase + kr, 16)][0]
                    pr = sv * CH + kr
                    xr = [buf[kr, pl.ds(16 * j, 16)]
                          + posbuf[pr, pl.ds(16 * j, 16)] for j in range(NV)]
                    yr, mr = _ln_stats(xr)
                    ks.append(kr)
                    xs.append(xr)
                    ys.append(yr)
                    minvs.append(mr)
                for j in range(NV):
                    sl = pl.ds(16 * j, 16)
                    g = gvec[sl]
                    b = bvec[sl]
                    for r in range(RG):
                        buf[ks[r], sl] = (xs[r][j] * ys[r] - minvs[r]) * g + b

        issue_gather(0, rows0, gsem0)
        issue_gather(1, rows1, gsem1)

        def stage(c, buf, gsem, wsem, prefetch):
            wait_gather(buf, gsem)
            ln_rows(buf, c)
            issue_write(c, buf, wsem)
            if prefetch:
                wait_write(buf, wsem)
                issue_gather(c + 2, buf, gsem)

        def pair(i, carry):
            stage(2 * i, rows0, gsem0, wsem0, True)
            stage(2 * i + 1, rows1, gsem1, wsem1, True)
            return carry

        lax.fori_loop(0, NCH // 2 - 1, pair, 0)
        stage(NCH - 2, rows0, gsem0, wsem0, False)
        stage(NCH - 1, rows1, gsem1, wsem1, False)
        wait_write(rows0, wsem0)
        wait_write(rows1, wsem1)

    return emb_kernel


def kernel(input_tensor, segment_tensor, token_table, segment_table,
           position_table, gamma, beta):
    B, L = input_tensor.shape
    V, D = token_table.shape
    N = B * L
    idx = input_tensor.reshape(N).astype(jnp.int32)
    sidx = segment_tensor.reshape(N).astype(jnp.int32)
    emb = _make_sc_kernel(N, D, L)
    out = emb(idx, sidx, token_table, segment_table, position_table,
              gamma, beta)
    return out.reshape(B, L, D)


# speedup vs baseline: 1.5349x; 1.0055x over previous
import functools

import jax
import jax.numpy as jnp
from jax import lax
from jax.experimental import pallas as pl
from jax.experimental.pallas import tpu as pltpu
from jax.experimental.pallas import tpu_sc as plsc


def _lane_permute(x, perm):
    dnums = lax.GatherDimensionNumbers(
        offset_dims=(), collapsed_slice_dims=(0,), start_index_map=(0,))
    return lax.gather(x, perm[:, None], dnums, (1,),
                      mode=lax.GatherScatterMode.PROMISE_IN_BOUNDS)


def _make_sc_kernel(N, D, CH):
    NW = 32
    TPW = N // NW
    NCH = TPW // CH
    NV = D // 16
    CH_A = 104
    CH_B = CH - CH_A

    mesh = plsc.VectorSubcoreMesh(core_axis_name="c", subcore_axis_name="s")

    @functools.partial(
        pl.kernel,
        mesh=mesh,
        out_type=jax.ShapeDtypeStruct((N, D), jnp.float32),
        scratch_types=[
            pltpu.VMEM((TPW,), jnp.int32),
            pltpu.VMEM((TPW + 16,), jnp.int32),
            pltpu.VMEM((2 * CH, D), jnp.float32),
            pltpu.VMEM((2, D), jnp.float32),
            pltpu.VMEM((CH, D), jnp.float32),
            pltpu.VMEM((CH, D), jnp.float32),
            pltpu.VMEM((D,), jnp.float32),
            pltpu.VMEM((D,), jnp.float32),
            pltpu.SemaphoreType.DMA,
            pltpu.SemaphoreType.DMA,
            pltpu.SemaphoreType.DMA,
            pltpu.SemaphoreType.DMA,
        ],
    )
    def emb_kernel(idx_hbm, seg_hbm, tok_hbm, segtab_hbm, pos_hbm,
                   g_hbm, b_hbm, out_hbm,
                   idx_all, seg_all, posbuf, segrows, rows0, rows1,
                   gvec, bvec, gsem0, gsem1, wsem0, wsem1):
        wid = lax.axis_index("s") * 2 + lax.axis_index("c")
        base = wid * TPW

        pltpu.sync_copy(idx_hbm.at[pl.ds(base, TPW)], idx_all)
        pltpu.sync_copy(seg_hbm.at[pl.ds(base, TPW)], seg_all.at[pl.ds(0, TPW)])
        pltpu.sync_copy(pos_hbm.at[pl.ds(0, CH)], posbuf.at[pl.ds(0, CH)])
        pltpu.sync_copy(pos_hbm.at[pl.ds(0, CH)], posbuf.at[pl.ds(CH, CH)])
        pltpu.sync_copy(segtab_hbm.at[pl.ds(0, 2)], segrows)
        pltpu.sync_copy(g_hbm, gvec)
        pltpu.sync_copy(b_hbm, bvec)

        @plsc.parallel_loop(0, CH, 1, unroll=2)
        def add_seg(r):
            for j in range(NV):
                sl = pl.ds(16 * j, 16)
                posbuf[r, sl] = posbuf[r, sl] + segrows[0, sl]
                posbuf[CH + r, sl] = posbuf[CH + r, sl] + segrows[1, sl]

        def issue_gather(c, buf, sem):
            pltpu.async_copy(tok_hbm.at[idx_all.at[pl.ds(c * CH, CH_A)]],
                             buf.at[pl.ds(0, CH_A)], sem)
            pltpu.async_copy(tok_hbm.at[idx_all.at[pl.ds(c * CH + CH_A, CH_B)]],
                             buf.at[pl.ds(CH_A, CH_B)], sem)

        def wait_gather(buf, sem):
            pltpu.make_async_copy(out_hbm.at[pl.ds(0, CH)], buf, sem).wait()

        def issue_write(c, buf, sem):
            pltpu.async_copy(buf, out_hbm.at[pl.ds(base + c * CH, CH)], sem)

        def wait_write(buf, sem):
            pltpu.make_async_copy(buf, out_hbm.at[pl.ds(0, CH)], sem).wait()

        def _ln_stats(xs):
            acc = xs[0]
            sq = xs[0] * xs[0]
            for j in range(1, NV):
                acc = acc + xs[j]
                sq = sq + xs[j] * xs[j]
            for s in (8, 4, 2, 1):
                perm = jnp.arange(16, dtype=jnp.int32) ^ s
                acc = acc + _lane_permute(acc, perm)
                sq = sq + _lane_permute(sq, perm)
            mean = acc * (1.0 / D)
            v = sq * (1.0 / D) - mean * mean + 1e-5
            iv = lax.bitcast_convert_type(v, jnp.int32)
            iv = jnp.int32(0x5F3759DF) - (iv >> 1)
            y = lax.bitcast_convert_type(iv, jnp.float32)
            vh = -0.5 * v
            for _ in range(1):
                y = y * (1.5 + vh * (y * y))
            return y, mean * y

        RG = 4

        def ln_rows(buf, c):
            cbase = c * CH

            @plsc.parallel_loop(0, CH, RG, unroll=2)
            def ln_row_group(k):
                ks, xs, ys, minvs = [], [], [], []
                for r in range(RG):
                    kr = k + r
                    sv = seg_all[pl.ds(cbase + kr, 16)][0]
                    pr = sv * CH + kr
                    xr = [buf[kr, pl.ds(16 * j, 16)]
                          + posbuf[pr, pl.ds(16 * j, 16)] for j in range(NV)]
                    yr, mr = _ln_stats(xr)
                    ks.append(kr)
                    xs.append(xr)
                    ys.append(yr)
                    minvs.append(mr)
                for j in range(NV):
                    sl = pl.ds(16 * j, 16)
                    g = gvec[sl]
                    b = bvec[sl]
                    for r in range(RG):
                        buf[ks[r], sl] = (xs[r][j] * ys[r] - minvs[r]) * g + b

        issue_gather(0, rows0, gsem0)
        issue_gather(1, rows1, gsem1)

        def stage(c, buf, gsem, wsem, prefetch):
            wait_gather(buf, gsem)
            ln_rows(buf, c)
            issue_write(c, buf, wsem)
            if prefetch:
                wait_write(buf, wsem)
                issue_gather(c + 2, buf, gsem)

        def pair(i, carry):
            stage(2 * i, rows0, gsem0, wsem0, True)
            stage(2 * i + 1, rows1, gsem1, wsem1, True)
            return carry

        lax.fori_loop(0, NCH // 2 - 1, pair, 0)
        stage(NCH - 2, rows0, gsem0, wsem0, False)
        stage(NCH - 1, rows1, gsem1, wsem1, False)
        wait_write(rows0, wsem0)
        wait_write(rows1, wsem1)

    return emb_kernel


def kernel(input_tensor, segment_tensor, token_table, segment_table,
           position_table, gamma, beta):
    B, L = input_tensor.shape
    V, D = token_table.shape
    N = B * L
    idx = input_tensor.reshape(N).astype(jnp.int32)
    sidx = segment_tensor.reshape(N).astype(jnp.int32)
    emb = _make_sc_kernel(N, D, L)
    out = emb(idx, sidx, token_table, segment_table, position_table,
              gamma, beta)
    return out.reshape(B, L, D)
